# baseline (speedup 1.0000x reference)
import functools

import jax
import jax.numpy as jnp
from jax import lax
from jax.experimental import pallas as pl
from jax.experimental.pallas import tpu as pltpu
from jax.experimental.pallas import tpu_sc as plsc

B, H, S, D, Q = 8, 8, 2048, 128, 1
HB = 4

NW = 32
ROWS = B * H * S
ROWS_PW = ROWS // NW
CH = 256
NCH = ROWS_PW // CH
NBUF = 3
RPW = (B * H) // NW
DUP = 8 // RPW


def _tc_k_body(pos_ref, k_ref, nk_ref, ko_ref):
    b = pl.program_id(0)
    p = pos_ref[b]
    ko_ref[...] = k_ref[...]
    ko_ref[0, :, p, :] = nk_ref[0, :, 0, :]


def _tc_k_update(k_cache, new_k, pos):
    cache_spec = pl.BlockSpec((1, HB, S, D), lambda b, h, pos_ref: (b, h, 0, 0))
    new_spec = pl.BlockSpec((1, HB, Q, D), lambda b, h, pos_ref: (b, h, 0, 0))
    grid_spec = pltpu.PrefetchScalarGridSpec(
        num_scalar_prefetch=1,
        grid=(B, H // HB),
        in_specs=[cache_spec, new_spec],
        out_specs=cache_spec,
    )
    return pl.pallas_call(
        _tc_k_body,
        grid_spec=grid_spec,
        out_shape=jax.ShapeDtypeStruct((B, H, S, D), k_cache.dtype),
    )(pos, k_cache, new_k)


def _sc_v_body(v_hbm, idx_hbm, rows_hbm, out_hbm,
               buf0, buf1, buf2, idx_v, rows_v, in_sem, out_sem, sc_sem):
    wid = lax.axis_index("s") * 2 + lax.axis_index("c")
    base = wid * ROWS_PW
    bufs = (buf0, buf1, buf2)
    in_copies = [None] * NBUF
    out_copies = [None] * NBUF
    for j in range(NBUF - 1):
        in_copies[j] = pltpu.async_copy(
            v_hbm.at[pl.ds(base + j * CH, CH)], bufs[j], in_sem.at[j])
    for i in range(NCH):
        cur = i % NBUF
        nxt = (i + NBUF - 1) % NBUF
        if i + NBUF - 1 < NCH:
            if out_copies[nxt] is not None:
                out_copies[nxt].wait()
            in_copies[nxt] = pltpu.async_copy(
                v_hbm.at[pl.ds(base + (i + NBUF - 1) * CH, CH)], bufs[nxt],
                in_sem.at[nxt])
        in_copies[cur].wait()
        out_copies[cur] = pltpu.async_copy(
            bufs[cur], out_hbm.at[pl.ds(base + i * CH, CH)], out_sem.at[cur])
    for j in range(NBUF):
        if out_copies[j] is not None:
            out_copies[j].wait()
    pltpu.sync_copy(idx_hbm.at[wid], idx_v)
    pltpu.sync_copy(rows_hbm.at[wid], rows_v)
    pltpu.async_copy(rows_v, out_hbm.at[idx_v], sc_sem).wait()


_sc_v_update = functools.partial(
    pl.kernel,
    out_type=jax.ShapeDtypeStruct((ROWS, D), jnp.float32),
    mesh=plsc.VectorSubcoreMesh(core_axis_name="c", subcore_axis_name="s"),
    scratch_types=[
        pltpu.VMEM((CH, D), jnp.float32),
        pltpu.VMEM((CH, D), jnp.float32),
        pltpu.VMEM((CH, D), jnp.float32),
        pltpu.VMEM((RPW * DUP,), jnp.int32),
        pltpu.VMEM((RPW * DUP, D), jnp.float32),
        pltpu.SemaphoreType.DMA((NBUF,)),
        pltpu.SemaphoreType.DMA((NBUF,)),
        pltpu.SemaphoreType.DMA,
    ],
)(_sc_v_body)


@jax.jit
def kernel(k_cache, v_cache, new_k, new_v, position_ids):
    pos = position_ids.reshape(B).astype(jnp.int32)

    rows = (jnp.arange(B, dtype=jnp.int32)[:, None] * (H * S)
            + jnp.arange(H, dtype=jnp.int32)[None, :] * S
            + pos[:, None])
    idx8 = jnp.tile(rows.reshape(NW, RPW), (1, DUP))
    rows8 = jnp.tile(new_v.reshape(NW, RPW, D), (1, DUP, 1))

    v_flat = v_cache.reshape(ROWS, D)
    v_out = _sc_v_update(v_flat, idx8, rows8).reshape(B, H, S, D)
    k_out = _tc_k_update(k_cache, new_k, pos)
    return (k_out, v_out)


# speedup vs baseline: 1.2875x; 1.2875x over previous
import jax
import jax.numpy as jnp
from jax.experimental import pallas as pl
from jax.experimental.pallas import tpu as pltpu

B, H, S, D, Q = 8, 8, 2048, 128, 1
HB = 4


def _update_body(pos_ref, k_ref, v_ref, nk_ref, nv_ref, ko_ref, vo_ref):
    b = pl.program_id(0)
    p = pos_ref[b]
    ko_ref[...] = k_ref[...]
    vo_ref[...] = v_ref[...]
    ko_ref[0, :, p, :] = nk_ref[0, :, 0, :]
    vo_ref[0, :, p, :] = nv_ref[0, :, 0, :]


@jax.jit
def kernel(k_cache, v_cache, new_k, new_v, position_ids):
    pos = position_ids.reshape(B)

    cache_spec = pl.BlockSpec((1, HB, S, D), lambda b, h, pos_ref: (b, h, 0, 0))
    new_spec = pl.BlockSpec((1, HB, Q, D), lambda b, h, pos_ref: (b, h, 0, 0))

    grid_spec = pltpu.PrefetchScalarGridSpec(
        num_scalar_prefetch=1,
        grid=(B, H // HB),
        in_specs=[cache_spec, cache_spec, new_spec, new_spec],
        out_specs=[cache_spec, cache_spec],
    )

    k_out, v_out = pl.pallas_call(
        _update_body,
        grid_spec=grid_spec,
        out_shape=[
            jax.ShapeDtypeStruct((B, H, S, D), k_cache.dtype),
            jax.ShapeDtypeStruct((B, H, S, D), v_cache.dtype),
        ],
    )(pos, k_cache, v_cache, new_k, new_v)
    return (k_out, v_out)
